# baseline (speedup 1.0000x reference)
import functools

import jax
import jax.numpy as jnp
from jax import lax
from jax.experimental import pallas as pl
from jax.experimental.pallas import tpu as pltpu
from jax.experimental.pallas import tpu_sc as plsc

D_MODEL = 768
D_FF = 1024
E = 64
TOPK = 2
T = 4096
A = T * TOPK
BLK = 512
NSTEPS = A // BLK + E
PAD = NSTEPS * BLK

NC, NS = 2, 16
NW = NC * NS
LANES = 16


ROUTER_TB = 512


def _router_body(x_ref, wg_ref, w_ref, idx_ref):
    logits = jnp.dot(x_ref[...], wg_ref[...], preferred_element_type=jnp.float32)
    col = lax.broadcasted_iota(jnp.int32, logits.shape, 1)
    m1 = jnp.max(logits, axis=-1, keepdims=True)
    i1 = jnp.min(jnp.where(logits == m1, col, E), axis=-1)
    rest = jnp.where(col == i1[:, None], -jnp.inf, logits)
    m2 = jnp.max(rest, axis=-1, keepdims=True)
    i2 = jnp.min(jnp.where(rest == m2, col, E), axis=-1)
    t = jnp.exp(m2[:, 0] - m1[:, 0])
    w1 = 1.0 / (1.0 + t)
    w_ref[...] = jnp.stack([w1, 1.0 - w1], axis=-1)
    idx_ref[...] = jnp.stack([i1, i2], axis=-1)


def _router(x_flat, Wg):
    return pl.pallas_call(
        _router_body,
        grid=(T // ROUTER_TB,),
        in_specs=[
            pl.BlockSpec((ROUTER_TB, D_MODEL), lambda i: (i, 0)),
            pl.BlockSpec((D_MODEL, E), lambda i: (0, 0)),
        ],
        out_specs=[
            pl.BlockSpec((ROUTER_TB, TOPK), lambda i: (i, 0)),
            pl.BlockSpec((ROUTER_TB, TOPK), lambda i: (i, 0)),
        ],
        out_shape=[
            jax.ShapeDtypeStruct((T, TOPK), jnp.float32),
            jax.ShapeDtypeStruct((T, TOPK), jnp.int32),
        ],
        compiler_params=pltpu.CompilerParams(
            dimension_semantics=("arbitrary",)),
    )(x_flat, Wg)


DISP_CHUNK = 64
DISP_NCH = A // DISP_CHUNK // NW


@functools.cache
def _dispatch_kernel():
    @functools.partial(
        pl.kernel,
        mesh=plsc.VectorSubcoreMesh(core_axis_name="c", subcore_axis_name="s"),
        out_type=jax.ShapeDtypeStruct((PAD, D_MODEL), jnp.float32),
        scratch_types=[
            pltpu.VMEM((DISP_CHUNK,), jnp.int32),
            pltpu.VMEM((DISP_CHUNK,), jnp.int32),
            pltpu.VMEM((DISP_CHUNK, D_MODEL), jnp.float32),
            pltpu.SemaphoreType.DMA,
        ],
    )
    def _dispatch(tid_hbm, pos_hbm, x_hbm, xg_hbm, tid_v, pos_v, rows_v, sem):
        wid = lax.axis_index("s") * NC + lax.axis_index("c")

        def body(i, _):
            base = (i * NW + wid) * DISP_CHUNK
            pltpu.sync_copy(tid_hbm.at[pl.ds(base, DISP_CHUNK)], tid_v)
            pltpu.sync_copy(pos_hbm.at[pl.ds(base, DISP_CHUNK)], pos_v)
            pltpu.async_copy(x_hbm.at[tid_v], rows_v, sem).wait()
            pltpu.sync_copy(rows_v, xg_hbm.at[pos_v])
            return 0

        lax.fori_loop(0, DISP_NCH, body, 0)

    return _dispatch


def _ffn_body(eob_ref, used_ref, blkmap_ref, xg_ref, w1_ref, b1_ref,
              w2_ref, b2_ref, out_ref):
    i = pl.program_id(0)

    @pl.when(used_ref[i] == 1)
    def _():
        xb = xg_ref[...].astype(jnp.bfloat16)
        w1b = w1_ref[0].astype(jnp.bfloat16)
        h = jnp.dot(xb, w1b, preferred_element_type=jnp.float32)
        h = jnp.maximum(h + b1_ref[0], 0.0)
        y = jnp.dot(h.astype(jnp.bfloat16), w2_ref[0].astype(jnp.bfloat16),
                    preferred_element_type=jnp.float32)
        out_ref[...] = y + b2_ref[0]


def _ffn(eob, used, blkmap, xg, W1, B1, W2, B2):
    grid_spec = pltpu.PrefetchScalarGridSpec(
        num_scalar_prefetch=3,
        grid=(NSTEPS,),
        in_specs=[
            pl.BlockSpec((BLK, D_MODEL), lambda i, eob, used, bm: (bm[i], 0)),
            pl.BlockSpec((1, D_MODEL, D_FF),
                         lambda i, eob, used, bm: (eob[i], 0, 0)),
            pl.BlockSpec((1, 1, D_FF), lambda i, eob, used, bm: (eob[i], 0, 0)),
            pl.BlockSpec((1, D_FF, D_MODEL),
                         lambda i, eob, used, bm: (eob[i], 0, 0)),
            pl.BlockSpec((1, 1, D_MODEL),
                         lambda i, eob, used, bm: (eob[i], 0, 0)),
        ],
        out_specs=pl.BlockSpec((BLK, D_MODEL), lambda i, eob, used, bm: (bm[i], 0)),
    )
    return pl.pallas_call(
        _ffn_body,
        grid_spec=grid_spec,
        out_shape=jax.ShapeDtypeStruct((PAD, D_MODEL), jnp.float32),
        compiler_params=pltpu.CompilerParams(
            dimension_semantics=("arbitrary",)),
    )(eob, used, blkmap, xg, W1, B1.reshape(E, 1, D_FF), W2,
      B2.reshape(E, 1, D_MODEL))


COMB_TOK = 32
TOK_PER_W = T // NW


@functools.cache
def _combine_kernel():
    @functools.partial(
        pl.kernel,
        mesh=plsc.VectorSubcoreMesh(core_axis_name="c", subcore_axis_name="s"),
        out_type=jax.ShapeDtypeStruct((T, D_MODEL), jnp.float32),
        scratch_types=[
            pltpu.VMEM((2 * COMB_TOK,), jnp.int32),
            pltpu.VMEM((2 * COMB_TOK + 16,), jnp.float32),
            pltpu.VMEM((2 * COMB_TOK, D_MODEL), jnp.float32),
            pltpu.VMEM((COMB_TOK, D_MODEL), jnp.float32),
            pltpu.SemaphoreType.DMA,
        ],
    )
    def _combine(pos_hbm, w_hbm, ygw_hbm, out_hbm, idx_v, w_v, rows_v, out_v,
                 sem):
        wid = lax.axis_index("s") * NC + lax.axis_index("c")

        def chunk(i, _):
            tbase = wid * TOK_PER_W + i * COMB_TOK
            pltpu.sync_copy(pos_hbm.at[pl.ds(tbase * 2, 2 * COMB_TOK)], idx_v)
            pltpu.sync_copy(w_hbm.at[pl.ds(tbase * 2, 2 * COMB_TOK)],
                            w_v.at[pl.ds(0, 2 * COMB_TOK)])
            pltpu.async_copy(ygw_hbm.at[idx_v], rows_v, sem).wait()

            def tok(t, _):
                wpair = w_v[pl.ds(2 * t, LANES)]
                w0 = wpair[0]
                w1 = wpair[1]
                for k in range(D_MODEL // LANES):
                    sl = pl.ds(k * LANES, LANES)
                    out_v[t, sl] = w0 * rows_v[2 * t, sl] + w1 * rows_v[2 * t + 1, sl]
                return 0

            lax.fori_loop(0, COMB_TOK, tok, 0)
            pltpu.sync_copy(out_v, out_hbm.at[pl.ds(tbase, COMB_TOK)])
            return 0

        lax.fori_loop(0, TOK_PER_W // COMB_TOK, chunk, 0)

    return _combine


NCHUNK = 64
CH = A // NCHUNK


def _plan(idx):
    eid = idx.reshape(-1)
    oh = (eid[:, None] == jnp.arange(E, dtype=jnp.int32)[None, :])
    oh_f = oh.astype(jnp.float32)
    oh3 = oh_f.reshape(NCHUNK, CH, E)
    ltri = jnp.tril(jnp.ones((CH, CH), jnp.float32), -1)
    within_excl = jnp.einsum("wv,cve->cwe", ltri, oh3)
    chunk_sums = jnp.sum(oh3, axis=1)
    chunk_prefix = jnp.cumsum(chunk_sums, axis=0) - chunk_sums
    rank = jnp.sum((chunk_prefix[:, None, :] + within_excl) * oh3,
                   axis=-1).reshape(A)
    counts = jnp.sum(oh_f, axis=0).astype(jnp.int32)
    pc = ((counts + BLK - 1) // BLK) * BLK
    cs_pc = jnp.cumsum(pc)
    pco = (cs_pc - pc).astype(jnp.float32)
    total = cs_pc[E - 1]
    pos_assign = (jnp.sum(oh_f * pco[None, :], axis=1) + rank).astype(jnp.int32)
    bstart = jnp.arange(NSTEPS, dtype=jnp.int32) * BLK
    used = (bstart < total).astype(jnp.int32)
    eob_used = jnp.sum((cs_pc[None, :] <= bstart[:, None]).astype(jnp.int32),
                       axis=1)
    last_e = jnp.sum((cs_pc <= total - 1).astype(jnp.int32))
    eob = jnp.where(used == 1, eob_used, last_e).astype(jnp.int32)
    nblocks_used = (total + BLK - 1) // BLK
    blkmap = jnp.where(used == 1, jnp.arange(NSTEPS, dtype=jnp.int32),
                       nblocks_used - 1).astype(jnp.int32)
    return pos_assign, eob, used, blkmap


def kernel(x, Wg, W1, B1, W2, B2):
    B, S, D = x.shape
    x_flat = x.reshape(-1, D)
    w, idx = _router(x_flat, Wg)
    pos_assign, eob, used, blkmap = _plan(idx)
    tid = jnp.repeat(jnp.arange(T, dtype=jnp.int32), TOPK)
    xg = _dispatch_kernel()(tid, pos_assign, x_flat)
    ygw = _ffn(eob, used, blkmap, xg, W1, B1, W2, B2)
    out = _combine_kernel()(pos_assign, w.reshape(-1), ygw)
    return out.reshape(B, S, D)


# speedup vs baseline: 1.1694x; 1.1694x over previous
import functools

import jax
import jax.numpy as jnp
from jax import lax
from jax.experimental import pallas as pl
from jax.experimental.pallas import tpu as pltpu
from jax.experimental.pallas import tpu_sc as plsc

D_MODEL = 768
D_FF = 1024
E = 64
TOPK = 2
T = 4096
A = T * TOPK
BLK = 256
NSTEPS = A // BLK + E
PAD = NSTEPS * BLK

NC, NS = 2, 16
NW = NC * NS
LANES = 16


ROUTER_TB = 512


def _router_body(x_ref, wg_ref, w_ref, idx_ref):
    logits = jnp.dot(x_ref[...], wg_ref[...], preferred_element_type=jnp.float32)
    col = lax.broadcasted_iota(jnp.int32, logits.shape, 1)
    m1 = jnp.max(logits, axis=-1, keepdims=True)
    i1 = jnp.min(jnp.where(logits == m1, col, E), axis=-1)
    rest = jnp.where(col == i1[:, None], -jnp.inf, logits)
    m2 = jnp.max(rest, axis=-1, keepdims=True)
    i2 = jnp.min(jnp.where(rest == m2, col, E), axis=-1)
    t = jnp.exp(m2[:, 0] - m1[:, 0])
    w1 = 1.0 / (1.0 + t)
    w_ref[...] = jnp.stack([w1, 1.0 - w1], axis=-1)
    idx_ref[...] = jnp.stack([i1, i2], axis=-1)


def _router(x_flat, Wg):
    return pl.pallas_call(
        _router_body,
        grid=(T // ROUTER_TB,),
        in_specs=[
            pl.BlockSpec((ROUTER_TB, D_MODEL), lambda i: (i, 0)),
            pl.BlockSpec((D_MODEL, E), lambda i: (0, 0)),
        ],
        out_specs=[
            pl.BlockSpec((ROUTER_TB, TOPK), lambda i: (i, 0)),
            pl.BlockSpec((ROUTER_TB, TOPK), lambda i: (i, 0)),
        ],
        out_shape=[
            jax.ShapeDtypeStruct((T, TOPK), jnp.float32),
            jax.ShapeDtypeStruct((T, TOPK), jnp.int32),
        ],
        compiler_params=pltpu.CompilerParams(
            dimension_semantics=("arbitrary",)),
    )(x_flat, Wg)


DISP_CHUNK = 64
DISP_NCH = A // DISP_CHUNK // NW


@functools.cache
def _dispatch_kernel():
    @functools.partial(
        pl.kernel,
        mesh=plsc.VectorSubcoreMesh(core_axis_name="c", subcore_axis_name="s"),
        out_type=jax.ShapeDtypeStruct((PAD, D_MODEL), jnp.float32),
        scratch_types=[
            pltpu.VMEM((DISP_CHUNK,), jnp.int32),
            pltpu.VMEM((DISP_CHUNK,), jnp.int32),
            pltpu.VMEM((DISP_CHUNK, D_MODEL), jnp.float32),
            pltpu.SemaphoreType.DMA,
        ],
    )
    def _dispatch(tid_hbm, pos_hbm, x_hbm, xg_hbm, tid_v, pos_v, rows_v, sem):
        wid = lax.axis_index("s") * NC + lax.axis_index("c")

        def body(i, _):
            base = (i * NW + wid) * DISP_CHUNK
            pltpu.sync_copy(tid_hbm.at[pl.ds(base, DISP_CHUNK)], tid_v)
            pltpu.sync_copy(pos_hbm.at[pl.ds(base, DISP_CHUNK)], pos_v)
            pltpu.async_copy(x_hbm.at[tid_v], rows_v, sem).wait()
            pltpu.sync_copy(rows_v, xg_hbm.at[pos_v])
            return 0

        lax.fori_loop(0, DISP_NCH, body, 0)

    return _dispatch


def _ffn_body(eob_ref, used_ref, blkmap_ref, xg_ref, w1_ref, b1_ref,
              w2_ref, b2_ref, out_ref):
    i = pl.program_id(0)

    @pl.when(used_ref[i] == 1)
    def _():
        xb = xg_ref[...].astype(jnp.bfloat16)
        w1b = w1_ref[0].astype(jnp.bfloat16)
        h = jnp.dot(xb, w1b, preferred_element_type=jnp.float32)
        h = jnp.maximum(h + b1_ref[0], 0.0)
        y = jnp.dot(h.astype(jnp.bfloat16), w2_ref[0].astype(jnp.bfloat16),
                    preferred_element_type=jnp.float32)
        out_ref[...] = y + b2_ref[0]


def _ffn(eob, used, blkmap, xg, W1, B1, W2, B2):
    grid_spec = pltpu.PrefetchScalarGridSpec(
        num_scalar_prefetch=3,
        grid=(NSTEPS,),
        in_specs=[
            pl.BlockSpec((BLK, D_MODEL), lambda i, eob, used, bm: (bm[i], 0)),
            pl.BlockSpec((1, D_MODEL, D_FF),
                         lambda i, eob, used, bm: (eob[i], 0, 0)),
            pl.BlockSpec((1, 1, D_FF), lambda i, eob, used, bm: (eob[i], 0, 0)),
            pl.BlockSpec((1, D_FF, D_MODEL),
                         lambda i, eob, used, bm: (eob[i], 0, 0)),
            pl.BlockSpec((1, 1, D_MODEL),
                         lambda i, eob, used, bm: (eob[i], 0, 0)),
        ],
        out_specs=pl.BlockSpec((BLK, D_MODEL), lambda i, eob, used, bm: (bm[i], 0)),
    )
    return pl.pallas_call(
        _ffn_body,
        grid_spec=grid_spec,
        out_shape=jax.ShapeDtypeStruct((PAD, D_MODEL), jnp.float32),
        compiler_params=pltpu.CompilerParams(
            dimension_semantics=("arbitrary",)),
    )(eob, used, blkmap, xg, W1, B1.reshape(E, 1, D_FF), W2,
      B2.reshape(E, 1, D_MODEL))


COMB_TOK = 32
TOK_PER_W = T // NW


@functools.cache
def _combine_kernel():
    NCHUNKS = TOK_PER_W // COMB_TOK

    @functools.partial(
        pl.kernel,
        mesh=plsc.VectorSubcoreMesh(core_axis_name="c", subcore_axis_name="s"),
        out_type=jax.ShapeDtypeStruct((T, D_MODEL), jnp.float32),
        scratch_types=[
            pltpu.VMEM((2 * COMB_TOK,), jnp.int32),
            pltpu.VMEM((2 * COMB_TOK,), jnp.int32),
            pltpu.VMEM((2 * COMB_TOK + 16,), jnp.float32),
            pltpu.VMEM((2 * COMB_TOK + 16,), jnp.float32),
            pltpu.VMEM((2 * COMB_TOK, D_MODEL), jnp.float32),
            pltpu.VMEM((2 * COMB_TOK, D_MODEL), jnp.float32),
            pltpu.VMEM((COMB_TOK, D_MODEL), jnp.float32),
            pltpu.SemaphoreType.DMA,
            pltpu.SemaphoreType.DMA,
        ],
    )
    def _combine(pos_hbm, w_hbm, ygw_hbm, out_hbm, idx0, idx1, w0v, w1v,
                 rows0, rows1, out_v, sem0, sem1):
        wid = lax.axis_index("s") * NC + lax.axis_index("c")
        idxs, wvs, rows, sems = (idx0, idx1), (w0v, w1v), (rows0, rows1), \
            (sem0, sem1)

        def start(i):
            tbase = wid * TOK_PER_W + i * COMB_TOK
            b = i % 2
            pltpu.sync_copy(pos_hbm.at[pl.ds(tbase * 2, 2 * COMB_TOK)], idxs[b])
            pltpu.sync_copy(w_hbm.at[pl.ds(tbase * 2, 2 * COMB_TOK)],
                            wvs[b].at[pl.ds(0, 2 * COMB_TOK)])
            return pltpu.async_copy(ygw_hbm.at[idxs[b]], rows[b], sems[b])

        g = start(0)
        for i in range(NCHUNKS):
            b = i % 2
            g_next = start(i + 1) if i + 1 < NCHUNKS else None
            g.wait()
            rows_v = rows[b]
            w_v = wvs[b]

            def tok(t, _):
                wpair = w_v[pl.ds(2 * t, LANES)]
                wa = wpair[0]
                wb = wpair[1]
                for k in range(D_MODEL // LANES):
                    sl = pl.ds(k * LANES, LANES)
                    out_v[t, sl] = (wa * rows_v[2 * t, sl]
                                    + wb * rows_v[2 * t + 1, sl])
                return 0

            lax.fori_loop(0, COMB_TOK, tok, 0)
            tbase = wid * TOK_PER_W + i * COMB_TOK
            pltpu.sync_copy(out_v, out_hbm.at[pl.ds(tbase, COMB_TOK)])
            g = g_next

    return _combine


NCHUNK = 64
CH = A // NCHUNK


def _plan(idx):
    eid = idx.reshape(-1)
    oh = (eid[:, None] == jnp.arange(E, dtype=jnp.int32)[None, :])
    oh_f = oh.astype(jnp.float32)
    oh3 = oh_f.reshape(NCHUNK, CH, E)
    ltri = jnp.tril(jnp.ones((CH, CH), jnp.float32), -1)
    within_excl = jnp.einsum("wv,cve->cwe", ltri, oh3)
    chunk_sums = jnp.sum(oh3, axis=1)
    chunk_prefix = jnp.cumsum(chunk_sums, axis=0) - chunk_sums
    rank = jnp.sum((chunk_prefix[:, None, :] + within_excl) * oh3,
                   axis=-1).reshape(A)
    counts = jnp.sum(oh_f, axis=0).astype(jnp.int32)
    pc = ((counts + BLK - 1) // BLK) * BLK
    cs_pc = jnp.cumsum(pc)
    pco = (cs_pc - pc).astype(jnp.float32)
    total = cs_pc[E - 1]
    pos_assign = (jnp.sum(oh_f * pco[None, :], axis=1) + rank).astype(jnp.int32)
    bstart = jnp.arange(NSTEPS, dtype=jnp.int32) * BLK
    used = (bstart < total).astype(jnp.int32)
    eob_used = jnp.sum((cs_pc[None, :] <= bstart[:, None]).astype(jnp.int32),
                       axis=1)
    last_e = jnp.sum((cs_pc <= total - 1).astype(jnp.int32))
    eob = jnp.where(used == 1, eob_used, last_e).astype(jnp.int32)
    nblocks_used = (total + BLK - 1) // BLK
    blkmap = jnp.where(used == 1, jnp.arange(NSTEPS, dtype=jnp.int32),
                       nblocks_used - 1).astype(jnp.int32)
    return pos_assign, eob, used, blkmap


def kernel(x, Wg, W1, B1, W2, B2):
    B, S, D = x.shape
    x_flat = x.reshape(-1, D)
    w, idx = _router(x_flat, Wg)
    pos_assign, eob, used, blkmap = _plan(idx)
    tid = jnp.repeat(jnp.arange(T, dtype=jnp.int32), TOPK)
    xg = _dispatch_kernel()(tid, pos_assign, x_flat)
    ygw = _ffn(eob, used, blkmap, xg, W1, B1, W2, B2)
    out = _combine_kernel()(pos_assign, w.reshape(-1), ygw)
    return out.reshape(B, S, D)


# speedup vs baseline: 1.1888x; 1.0166x over previous
import functools

import jax
import jax.numpy as jnp
from jax import lax
from jax.experimental import pallas as pl
from jax.experimental.pallas import tpu as pltpu
from jax.experimental.pallas import tpu_sc as plsc

D_MODEL = 768
D_FF = 1024
E = 64
TOPK = 2
T = 4096
A = T * TOPK
BLK = 256
NSTEPS = A // BLK + E
PAD = NSTEPS * BLK

NC, NS = 2, 16
NW = NC * NS
LANES = 16


ROUTER_TB = 512


def _router_body(x_ref, wg_ref, w_ref, idx_ref):
    logits = jnp.dot(x_ref[...], wg_ref[...], preferred_element_type=jnp.float32)
    col = lax.broadcasted_iota(jnp.int32, logits.shape, 1)
    m1 = jnp.max(logits, axis=-1, keepdims=True)
    i1 = jnp.min(jnp.where(logits == m1, col, E), axis=-1)
    rest = jnp.where(col == i1[:, None], -jnp.inf, logits)
    m2 = jnp.max(rest, axis=-1, keepdims=True)
    i2 = jnp.min(jnp.where(rest == m2, col, E), axis=-1)
    t = jnp.exp(m2[:, 0] - m1[:, 0])
    w1 = 1.0 / (1.0 + t)
    w_ref[...] = jnp.stack([w1, 1.0 - w1], axis=-1)
    idx_ref[...] = jnp.stack([i1, i2], axis=-1)


def _router(x_flat, Wg):
    return pl.pallas_call(
        _router_body,
        grid=(T // ROUTER_TB,),
        in_specs=[
            pl.BlockSpec((ROUTER_TB, D_MODEL), lambda i: (i, 0)),
            pl.BlockSpec((D_MODEL, E), lambda i: (0, 0)),
        ],
        out_specs=[
            pl.BlockSpec((ROUTER_TB, TOPK), lambda i: (i, 0)),
            pl.BlockSpec((ROUTER_TB, TOPK), lambda i: (i, 0)),
        ],
        out_shape=[
            jax.ShapeDtypeStruct((T, TOPK), jnp.float32),
            jax.ShapeDtypeStruct((T, TOPK), jnp.int32),
        ],
        compiler_params=pltpu.CompilerParams(
            dimension_semantics=("arbitrary",)),
    )(x_flat, Wg)


DISP_TOK = 64
DISP_NCH = T // DISP_TOK // NW


@functools.cache
def _dispatch_kernel():
    @functools.partial(
        pl.kernel,
        mesh=plsc.VectorSubcoreMesh(core_axis_name="c", subcore_axis_name="s"),
        out_type=jax.ShapeDtypeStruct((PAD, D_MODEL), jnp.float32),
        scratch_types=[
            pltpu.VMEM((DISP_TOK,), jnp.int32),
            pltpu.VMEM((DISP_TOK,), jnp.int32),
            pltpu.VMEM((DISP_TOK, D_MODEL), jnp.float32),
            pltpu.SemaphoreType.DMA,
            pltpu.SemaphoreType.DMA,
        ],
    )
    def _dispatch(pe_hbm, po_hbm, x_hbm, xg_hbm, pe_v, po_v, rows_v,
                  sem0, sem1):
        wid = lax.axis_index("s") * NC + lax.axis_index("c")

        def body(i, _):
            tbase = (i * NW + wid) * DISP_TOK
            pltpu.sync_copy(pe_hbm.at[pl.ds(tbase, DISP_TOK)], pe_v)
            pltpu.sync_copy(po_hbm.at[pl.ds(tbase, DISP_TOK)], po_v)
            pltpu.sync_copy(x_hbm.at[pl.ds(tbase, DISP_TOK)], rows_v)
            c0 = pltpu.async_copy(rows_v, xg_hbm.at[pe_v], sem0)
            c1 = pltpu.async_copy(rows_v, xg_hbm.at[po_v], sem1)
            c0.wait()
            c1.wait()
            return 0

        lax.fori_loop(0, DISP_NCH, body, 0)

    return _dispatch


def _ffn_body(eob_ref, used_ref, blkmap_ref, xg_ref, w1_ref, b1_ref,
              w2_ref, b2_ref, out_ref):
    i = pl.program_id(0)

    @pl.when(used_ref[i] == 1)
    def _():
        xb = xg_ref[...].astype(jnp.bfloat16)
        w1b = w1_ref[0].astype(jnp.bfloat16)
        h = jnp.dot(xb, w1b, preferred_element_type=jnp.float32)
        h = jnp.maximum(h + b1_ref[0], 0.0)
        y = jnp.dot(h.astype(jnp.bfloat16), w2_ref[0].astype(jnp.bfloat16),
                    preferred_element_type=jnp.float32)
        out_ref[...] = y + b2_ref[0]


def _ffn(eob, used, blkmap, xg, W1, B1, W2, B2):
    grid_spec = pltpu.PrefetchScalarGridSpec(
        num_scalar_prefetch=3,
        grid=(NSTEPS,),
        in_specs=[
            pl.BlockSpec((BLK, D_MODEL), lambda i, eob, used, bm: (bm[i], 0)),
            pl.BlockSpec((1, D_MODEL, D_FF),
                         lambda i, eob, used, bm: (eob[i], 0, 0)),
            pl.BlockSpec((1, 1, D_FF), lambda i, eob, used, bm: (eob[i], 0, 0)),
            pl.BlockSpec((1, D_FF, D_MODEL),
                         lambda i, eob, used, bm: (eob[i], 0, 0)),
            pl.BlockSpec((1, 1, D_MODEL),
                         lambda i, eob, used, bm: (eob[i], 0, 0)),
        ],
        out_specs=pl.BlockSpec((BLK, D_MODEL), lambda i, eob, used, bm: (bm[i], 0)),
    )
    return pl.pallas_call(
        _ffn_body,
        grid_spec=grid_spec,
        out_shape=jax.ShapeDtypeStruct((PAD, D_MODEL), jnp.float32),
        compiler_params=pltpu.CompilerParams(
            dimension_semantics=("arbitrary",)),
    )(eob, used, blkmap, xg, W1, B1.reshape(E, 1, D_FF), W2,
      B2.reshape(E, 1, D_MODEL))


COMB_TOK = 32
TOK_PER_W = T // NW


@functools.cache
def _combine_kernel():
    NCHUNKS = TOK_PER_W // COMB_TOK

    @functools.partial(
        pl.kernel,
        mesh=plsc.VectorSubcoreMesh(core_axis_name="c", subcore_axis_name="s"),
        out_type=jax.ShapeDtypeStruct((T, D_MODEL), jnp.float32),
        scratch_types=[
            pltpu.VMEM((2 * COMB_TOK,), jnp.int32),
            pltpu.VMEM((2 * COMB_TOK,), jnp.int32),
            pltpu.VMEM((2 * COMB_TOK + 16,), jnp.float32),
            pltpu.VMEM((2 * COMB_TOK + 16,), jnp.float32),
            pltpu.VMEM((2 * COMB_TOK, D_MODEL), jnp.float32),
            pltpu.VMEM((2 * COMB_TOK, D_MODEL), jnp.float32),
            pltpu.VMEM((COMB_TOK, D_MODEL), jnp.float32),
            pltpu.SemaphoreType.DMA,
            pltpu.SemaphoreType.DMA,
        ],
    )
    def _combine(pos_hbm, w_hbm, ygw_hbm, out_hbm, idx0, idx1, w0v, w1v,
                 rows0, rows1, out_v, sem0, sem1):
        wid = lax.axis_index("s") * NC + lax.axis_index("c")
        idxs, wvs, rows, sems = (idx0, idx1), (w0v, w1v), (rows0, rows1), \
            (sem0, sem1)

        def start(i):
            tbase = wid * TOK_PER_W + i * COMB_TOK
            b = i % 2
            pltpu.sync_copy(pos_hbm.at[pl.ds(tbase * 2, 2 * COMB_TOK)], idxs[b])
            pltpu.sync_copy(w_hbm.at[pl.ds(tbase * 2, 2 * COMB_TOK)],
                            wvs[b].at[pl.ds(0, 2 * COMB_TOK)])
            return pltpu.async_copy(ygw_hbm.at[idxs[b]], rows[b], sems[b])

        g = start(0)
        for i in range(NCHUNKS):
            b = i % 2
            g_next = start(i + 1) if i + 1 < NCHUNKS else None
            g.wait()
            rows_v = rows[b]
            w_v = wvs[b]

            def tok(t, _):
                wpair = w_v[pl.ds(2 * t, LANES)]
                wa = wpair[0]
                wb = wpair[1]
                for k in range(D_MODEL // LANES):
                    sl = pl.ds(k * LANES, LANES)
                    out_v[t, sl] = (wa * rows_v[2 * t, sl]
                                    + wb * rows_v[2 * t + 1, sl])
                return 0

            lax.fori_loop(0, COMB_TOK, tok, 0)
            tbase = wid * TOK_PER_W + i * COMB_TOK
            pltpu.sync_copy(out_v, out_hbm.at[pl.ds(tbase, COMB_TOK)])
            g = g_next

    return _combine


NCHUNK = 64
CH = A // NCHUNK


def _plan(idx):
    eid = idx.reshape(-1)
    oh = (eid[:, None] == jnp.arange(E, dtype=jnp.int32)[None, :])
    oh_f = oh.astype(jnp.float32)
    oh3 = oh_f.reshape(NCHUNK, CH, E)
    ltri = jnp.tril(jnp.ones((CH, CH), jnp.float32), -1)
    within_excl = jnp.einsum("wv,cve->cwe", ltri, oh3)
    chunk_sums = jnp.sum(oh3, axis=1)
    chunk_prefix = jnp.cumsum(chunk_sums, axis=0) - chunk_sums
    rank = jnp.sum((chunk_prefix[:, None, :] + within_excl) * oh3,
                   axis=-1).reshape(A)
    counts = jnp.sum(oh_f, axis=0).astype(jnp.int32)
    pc = ((counts + BLK - 1) // BLK) * BLK
    cs_pc = jnp.cumsum(pc)
    pco = (cs_pc - pc).astype(jnp.float32)
    total = cs_pc[E - 1]
    pos_assign = (jnp.sum(oh_f * pco[None, :], axis=1) + rank).astype(jnp.int32)
    bstart = jnp.arange(NSTEPS, dtype=jnp.int32) * BLK
    used = (bstart < total).astype(jnp.int32)
    eob_used = jnp.sum((cs_pc[None, :] <= bstart[:, None]).astype(jnp.int32),
                       axis=1)
    last_e = jnp.sum((cs_pc <= total - 1).astype(jnp.int32))
    eob = jnp.where(used == 1, eob_used, last_e).astype(jnp.int32)
    nblocks_used = (total + BLK - 1) // BLK
    blkmap = jnp.where(used == 1, jnp.arange(NSTEPS, dtype=jnp.int32),
                       nblocks_used - 1).astype(jnp.int32)
    return pos_assign, eob, used, blkmap


def kernel(x, Wg, W1, B1, W2, B2):
    B, S, D = x.shape
    x_flat = x.reshape(-1, D)
    w, idx = _router(x_flat, Wg)
    pos_assign, eob, used, blkmap = _plan(idx)
    pos2 = pos_assign.reshape(T, TOPK)
    xg = _dispatch_kernel()(pos2[:, 0], pos2[:, 1], x_flat)
    ygw = _ffn(eob, used, blkmap, xg, W1, B1, W2, B2)
    out = _combine_kernel()(pos_assign, w.reshape(-1), ygw)
    return out.reshape(B, S, D)
